# Initial kernel scaffold; baseline (speedup 1.0000x reference)
import jax
import jax.numpy as jnp
from jax.experimental import pallas as pl


def kernel(z, u, w, W_mu, b_mu, W_lv, b_lv):
    raise NotImplementedError("write your pallas kernel here")


# speedup vs baseline: 1.6293x; 1.6293x over previous
import functools

import jax
import jax.numpy as jnp
from jax.experimental import pallas as pl
from jax.experimental.pallas import tpu as pltpu

_L = 64
_HALF_LOG_2PI = 0.9189385332046727
_KB = 8


def _tc_body(z2_ref, u_ref, w_ref, Wmu_ref, bmu_ref, Wlv_ref, blv_ref,
             out_ref, c0_ref, c1_ref, c2_ref):
    K = u_ref.shape[0]
    mean = jnp.dot(u_ref[...], Wmu_ref[...],
                   preferred_element_type=jnp.float32) + bmu_ref[...]
    lv = jnp.dot(u_ref[...], Wlv_ref[...],
                 preferred_element_type=jnp.float32) + blv_ref[...]
    wv = w_ref[...]
    wmax = jnp.max(wv)
    logw = wv - (wmax + jnp.log(jnp.sum(jnp.exp(wv - wmax))))
    t = logw - 0.5 * lv - _HALF_LOG_2PI
    m_l = jnp.max(t, axis=0, keepdims=True)
    p = 0.5 * jnp.exp(-lv)
    c0 = (t - m_l) - p * mean * mean
    c1 = 2.0 * p * mean
    c2 = -p
    c0_ref[...] = jnp.concatenate([c0, c0], axis=1)
    c1_ref[...] = jnp.concatenate([c1, c1], axis=1)
    c2_ref[...] = jnp.concatenate([c2, c2], axis=1)

    z = z2_ref[...]
    zz = z * z

    def body(i, s):
        base = i * _KB
        r0 = c0_ref[pl.ds(base, _KB), :]
        r1 = c1_ref[pl.ds(base, _KB), :]
        r2 = c2_ref[pl.ds(base, _KB), :]
        for j in range(_KB):
            arg = r0[j:j + 1, :] + r1[j:j + 1, :] * z + r2[j:j + 1, :] * zz
            s = s + jnp.exp(arg)
        return s

    s = jax.lax.fori_loop(0, K // _KB, body,
                          jnp.zeros(z2_ref.shape, jnp.float32))
    m_t = jnp.concatenate([m_l, m_l], axis=1)
    out_ref[...] = m_t + jnp.log(s)


@jax.jit
def kernel(z, u, w, W_mu, b_mu, W_lv, b_lv):
    B, L = z.shape
    K = u.shape[0]
    z2 = z.reshape(B // 2, 2 * L)
    out2 = pl.pallas_call(
        _tc_body,
        out_shape=jax.ShapeDtypeStruct((B // 2, 2 * L), jnp.float32),
        scratch_shapes=[
            pltpu.VMEM((K, 2 * L), jnp.float32),
            pltpu.VMEM((K, 2 * L), jnp.float32),
            pltpu.VMEM((K, 2 * L), jnp.float32),
        ],
    )(z2, u, w.reshape(K, 1), W_mu, b_mu.reshape(1, L), W_lv,
      b_lv.reshape(1, L))
    return out2.reshape(B, L)


# speedup vs baseline: 1.8420x; 1.1305x over previous
import functools

import jax
import jax.numpy as jnp
from jax.experimental import pallas as pl
from jax.experimental.pallas import tpu as pltpu

_L = 64
_HALF_LOG_2PI = 0.9189385332046727
_KB = 4
_RB = 128


def _tc_body(z2_ref, u_ref, w_ref, Wmu_ref, bmu_ref, Wlv_ref, blv_ref,
             out_ref, c0_ref, c1_ref, c2_ref):
    K = u_ref.shape[0]
    mean = jnp.dot(u_ref[...], Wmu_ref[...],
                   preferred_element_type=jnp.float32) + bmu_ref[...]
    lv = jnp.dot(u_ref[...], Wlv_ref[...],
                 preferred_element_type=jnp.float32) + blv_ref[...]
    wv = w_ref[...]
    wmax = jnp.max(wv)
    logw = wv - (wmax + jnp.log(jnp.sum(jnp.exp(wv - wmax))))
    t = logw - 0.5 * lv - _HALF_LOG_2PI
    m_l = jnp.max(t, axis=0, keepdims=True)
    p = 0.5 * jnp.exp(-lv)
    c0 = (t - m_l) - p * mean * mean
    c1 = 2.0 * p * mean
    c2 = -p
    c0_ref[...] = jnp.concatenate([c0, c0], axis=1)
    c1_ref[...] = jnp.concatenate([c1, c1], axis=1)
    c2_ref[...] = jnp.concatenate([c2, c2], axis=1)

    m_t = jnp.concatenate([m_l, m_l], axis=1)
    nrows = z2_ref.shape[0]
    for c in range(nrows // _RB):
        z = z2_ref[c * _RB:(c + 1) * _RB, :]

        def body(i, s):
            base = i * _KB
            r0 = c0_ref[pl.ds(base, _KB), :]
            r1 = c1_ref[pl.ds(base, _KB), :]
            r2 = c2_ref[pl.ds(base, _KB), :]
            for j in range(_KB):
                t = r2[j:j + 1, :] * z + r1[j:j + 1, :]
                arg = t * z + r0[j:j + 1, :]
                s = s + jnp.exp(arg)
            return s

        s = jax.lax.fori_loop(0, K // _KB, body,
                              jnp.zeros((_RB, 2 * _L), jnp.float32))
        out_ref[c * _RB:(c + 1) * _RB, :] = m_t + jnp.log(s)


@jax.jit
def kernel(z, u, w, W_mu, b_mu, W_lv, b_lv):
    B, L = z.shape
    K = u.shape[0]
    z2 = z.reshape(B // 2, 2 * L)
    out2 = pl.pallas_call(
        _tc_body,
        out_shape=jax.ShapeDtypeStruct((B // 2, 2 * L), jnp.float32),
        scratch_shapes=[
            pltpu.VMEM((K, 2 * L), jnp.float32),
            pltpu.VMEM((K, 2 * L), jnp.float32),
            pltpu.VMEM((K, 2 * L), jnp.float32),
        ],
    )(z2, u, w.reshape(K, 1), W_mu, b_mu.reshape(1, L), W_lv,
      b_lv.reshape(1, L))
    return out2.reshape(B, L)
